# Initial kernel scaffold; baseline (speedup 1.0000x reference)
import jax
import jax.numpy as jnp
from jax.experimental import pallas as pl


def kernel(tables, W1, b1, W2, b2, W3, b3):
    raise NotImplementedError("write your pallas kernel here")


# speedup vs baseline: 30.9132x; 30.9132x over previous
import functools

import numpy as np
import jax
import jax.numpy as jnp
from jax import lax
from jax.experimental import pallas as pl
from jax.experimental.pallas import tpu as pltpu
from jax.experimental.pallas import tpu_sc as plsc

L = 16
W_RES = 1024
H_RES = 1024
HW = W_RES * H_RES
LOG2_T = 19
T_SIZE = 2 ** LOG2_T
F_DIM = 2
HIDDEN = 64
HASH_K = np.uint32(2654435761)
HASH_MASK = np.uint32(T_SIZE - 1)

NC, NS = 2, 16
NW = NC * NS
CHUNK = 2048
IDX_ROWS = CHUNK // 128


def _resolutions_np():
    b = np.exp((np.log(2048.0) - np.log(16.0)) / (L - 1))
    return np.floor(16.0 * (b ** np.arange(L))).astype(np.float32)


@functools.lru_cache(maxsize=1)
def _precompute():
    res = _resolutions_np()
    px = (np.arange(W_RES, dtype=np.float32) / np.float32(W_RES))
    idx_segs, amats, ns, offs = [], [], [], []
    off = 0
    for l in range(L - 1):
        r = np.float32(res[l])
        scaled = (px * r).astype(np.float32)
        pos = np.floor(scaled).astype(np.float32)
        ix = pos.astype(np.uint32)
        fx = (scaled - pos).astype(np.float32)
        n = int(ix.max()) + 2
        g = np.arange(n, dtype=np.uint32)
        hy = (g * HASH_K) & HASH_MASK
        grid = (g[None, :] ^ hy[:, None]) & HASH_MASK
        idx_segs.append((grid.astype(np.int64) + l * T_SIZE)
                        .astype(np.int32).reshape(-1))
        a = np.zeros((W_RES, n), np.float32)
        a[np.arange(W_RES), ix] += (1.0 - fx)
        a[np.arange(W_RES), ix + 1] += fx
        amats.append(a)
        ns.append(n)
        offs.append(off)
        off += n * n
    ix15 = (px * np.float32(res[L - 1])).astype(np.uint32)
    h15 = (ix15[None, :] ^ ((ix15[:, None] * HASH_K) & HASH_MASK)) & HASH_MASK
    idx_segs.append((h15.astype(np.int64) + (L - 1) * T_SIZE)
                    .astype(np.int32).reshape(-1))
    off15 = off
    off += HW
    nv = off
    group = NW * CHUNK
    nv_pad = ((nv + group - 1) // group) * group
    idx = np.concatenate(idx_segs)
    pad = (np.arange(nv_pad - nv, dtype=np.int64) % (L * T_SIZE)).astype(np.int32)
    idx = np.concatenate([idx, pad]).reshape(nv_pad // 128, 128)
    return dict(idx=idx, amats=amats, ns=ns, offs=offs, off15=off15,
                nv_pad=nv_pad)


def _sc_gather(tbl_flat, idx_np, nv_pad):
    per_w = nv_pad // NW
    n_outer = per_w // CHUNK
    rows_per_w = per_w // 128
    mesh = plsc.VectorSubcoreMesh(core_axis_name="c", subcore_axis_name="s",
                                  num_cores=NC, num_subcores=NS)

    @functools.partial(
        pl.kernel,
        out_type=jax.ShapeDtypeStruct((nv_pad, F_DIM), jnp.float32),
        mesh=mesh,
        scratch_types=[
            pltpu.VMEM((IDX_ROWS, 128), jnp.int32),
            pltpu.VMEM((CHUNK, F_DIM), jnp.float32),
            pltpu.SemaphoreType.DMA,
        ],
        compiler_params=pltpu.CompilerParams(use_tc_tiling_on_sc=False),
    )
    def gather_k(tbl_hbm, idx_hbm, out_hbm, idx_v, rows_v, sem):
        wid = lax.axis_index("s") * NC + lax.axis_index("c")
        base_idx_row = wid * rows_per_w
        base_out = wid * per_w

        def body(j, carry):
            row0 = base_idx_row + j * IDX_ROWS
            pltpu.sync_copy(idx_hbm.at[pl.ds(row0, IDX_ROWS)], idx_v)
            copies = [
                pltpu.async_copy(tbl_hbm.at[idx_v.at[t]],
                                 rows_v.at[pl.ds(t * 128, 128)], sem)
                for t in range(IDX_ROWS)
            ]
            for c in copies:
                c.wait()
            pltpu.sync_copy(rows_v,
                            out_hbm.at[pl.ds(base_out + j * CHUNK, CHUNK)])
            return carry

        lax.fori_loop(0, n_outer, body, 0)

    return gather_k(tbl_flat, jnp.asarray(idx_np))


def _level_interp(a_np, g0, g1):
    n = a_np.shape[1]
    a = jnp.asarray(a_np.astype(np.float32)).astype(jnp.bfloat16)
    at = jnp.asarray(np.ascontiguousarray(a_np.T).astype(np.float32)).astype(jnp.bfloat16)

    def body(a_ref, at_ref, g0_ref, g1_ref, o_ref):
        av = a_ref[...]
        atv = at_ref[...]
        for c, gref in ((0, g0_ref), (1, g1_ref)):
            t = jnp.dot(gref[...], atv, preferred_element_type=jnp.float32)
            e = jnp.dot(av, t.astype(jnp.bfloat16),
                        preferred_element_type=jnp.float32)
            o_ref[c] = e.astype(jnp.bfloat16)

    return pl.pallas_call(
        body,
        out_shape=jax.ShapeDtypeStruct((2, H_RES, W_RES), jnp.bfloat16),
    )(a, at, g0, g1)


def _mlp(enc, w1t, b1, w2t, b2, w3t, b3):
    bn = 8192
    grid = (HW // bn,)

    def body(x_ref, w1_ref, b1_ref, w2_ref, b2_ref, w3_ref, b3_ref, o_ref):
        x = x_ref[...]
        h = jnp.dot(w1_ref[...], x, preferred_element_type=jnp.float32)
        h = jnp.maximum(h + b1_ref[...], 0.0).astype(jnp.bfloat16)
        h = jnp.dot(w2_ref[...], h, preferred_element_type=jnp.float32)
        h = jnp.maximum(h + b2_ref[...], 0.0).astype(jnp.bfloat16)
        o = jnp.dot(w3_ref[...], h, preferred_element_type=jnp.float32)
        o_ref[...] = jax.nn.sigmoid(o + b3_ref[...])

    full = lambda s: pl.BlockSpec(s, lambda i: (0, 0))
    return pl.pallas_call(
        body,
        grid=grid,
        in_specs=[
            pl.BlockSpec((32, bn), lambda i: (0, i)),
            full((HIDDEN, 32)), full((HIDDEN, 1)),
            full((HIDDEN, HIDDEN)), full((HIDDEN, 1)),
            full((3, HIDDEN)), full((3, 1)),
        ],
        out_specs=pl.BlockSpec((3, bn), lambda i: (0, i)),
        out_shape=jax.ShapeDtypeStruct((3, HW), jnp.float32),
    )(enc, w1t, b1, w2t, b2, w3t, b3)


def kernel(tables, W1, b1, W2, b2, W3, b3):
    pre = _precompute()
    tbl = tables.reshape(L * T_SIZE, F_DIM)
    g = _sc_gather(tbl, pre["idx"], pre["nv_pad"])
    gb = g.astype(jnp.bfloat16)

    planes = []
    for l in range(L - 1):
        n, off = pre["ns"][l], pre["offs"][l]
        seg = lax.slice(gb, (off, 0), (off + n * n, F_DIM))
        g0 = seg[:, 0].reshape(n, n)
        g1 = seg[:, 1].reshape(n, n)
        planes.append(_level_interp(pre["amats"][l], g0, g1).reshape(2, HW))
    seg15 = lax.slice(gb, (pre["off15"], 0), (pre["off15"] + HW, F_DIM))
    planes.append(seg15.T)
    enc = jnp.concatenate(planes, axis=0)

    out = _mlp(enc,
               W1.T.astype(jnp.bfloat16), b1.reshape(HIDDEN, 1),
               W2.T.astype(jnp.bfloat16), b2.reshape(HIDDEN, 1),
               W3.T.astype(jnp.bfloat16), b3.reshape(3, 1))
    return out.reshape(1, 3, H_RES, W_RES)


# speedup vs baseline: 33.6554x; 1.0887x over previous
import functools

import numpy as np
import jax
import jax.numpy as jnp
from jax import lax
from jax.experimental import pallas as pl
from jax.experimental.pallas import tpu as pltpu
from jax.experimental.pallas import tpu_sc as plsc

L = 16
W_RES = 1024
H_RES = 1024
HW = W_RES * H_RES
LOG2_T = 19
T_SIZE = 2 ** LOG2_T
F_DIM = 2
HIDDEN = 64
HASH_K = np.uint32(2654435761)
HASH_MASK = np.uint32(T_SIZE - 1)

NC, NS = 2, 16
NW = NC * NS
CHUNK = 2048
IDX_ROWS = CHUNK // 128


def _resolutions_np():
    b = np.exp((np.log(2048.0) - np.log(16.0)) / (L - 1))
    return np.floor(16.0 * (b ** np.arange(L))).astype(np.float32)


@functools.lru_cache(maxsize=1)
def _precompute():
    res = _resolutions_np()
    px = (np.arange(W_RES, dtype=np.float32) / np.float32(W_RES))
    idx_segs, amats, ns, offs = [], [], [], []
    off = 0
    for l in range(L - 1):
        r = np.float32(res[l])
        scaled = (px * r).astype(np.float32)
        pos = np.floor(scaled).astype(np.float32)
        ix = pos.astype(np.uint32)
        fx = (scaled - pos).astype(np.float32)
        n = int(ix.max()) + 2
        g = np.arange(n, dtype=np.uint32)
        hy = (g * HASH_K) & HASH_MASK
        grid = (g[None, :] ^ hy[:, None]) & HASH_MASK
        idx_segs.append((grid.astype(np.int64) + l * T_SIZE)
                        .astype(np.int32).reshape(-1))
        a = np.zeros((W_RES, n), np.float32)
        a[np.arange(W_RES), ix] += (1.0 - fx)
        a[np.arange(W_RES), ix + 1] += fx
        amats.append(a)
        ns.append(n)
        offs.append(off)
        off += n * n
    ix15 = (px * np.float32(res[L - 1])).astype(np.uint32)
    h15 = (ix15[None, :] ^ ((ix15[:, None] * HASH_K) & HASH_MASK)) & HASH_MASK
    idx_segs.append((h15.astype(np.int64) + (L - 1) * T_SIZE)
                    .astype(np.int32).reshape(-1))
    off15 = off
    off += HW
    nv = off
    group = NW * CHUNK
    nv_pad = ((nv + group - 1) // group) * group
    idx = np.concatenate(idx_segs)
    pad = (np.arange(nv_pad - nv, dtype=np.int64) % (L * T_SIZE)).astype(np.int32)
    idx = np.concatenate([idx, pad]).reshape(nv_pad // 128, 128)
    return dict(idx=idx, amats=amats, ns=ns, offs=offs, off15=off15,
                nv_pad=nv_pad)


def _sc_gather(tbl_flat, idx_np, nv_pad):
    per_w = nv_pad // NW
    n_outer = per_w // CHUNK
    rows_per_w = per_w // 128
    mesh = plsc.VectorSubcoreMesh(core_axis_name="c", subcore_axis_name="s",
                                  num_cores=NC, num_subcores=NS)

    @functools.partial(
        pl.kernel,
        out_type=jax.ShapeDtypeStruct((F_DIM, nv_pad), jnp.float32),
        mesh=mesh,
        scratch_types=[
            pltpu.VMEM((IDX_ROWS, 128), jnp.int32),
            pltpu.VMEM((CHUNK, F_DIM), jnp.float32),
            pltpu.VMEM((CHUNK,), jnp.float32),
            pltpu.VMEM((CHUNK,), jnp.float32),
            pltpu.SemaphoreType.DMA,
        ],
        compiler_params=pltpu.CompilerParams(use_tc_tiling_on_sc=False,
                                             needs_layout_passes=False),
    )
    def gather_k(tbl_hbm, idx_hbm, out_hbm, idx_v, rows_v, c0_v, c1_v, sem):
        wid = lax.axis_index("s") * NC + lax.axis_index("c")
        base_idx_row = wid * rows_per_w
        base_out = wid * per_w
        ri = lax.iota(jnp.int32, 16)
        zeros16 = jnp.zeros((16,), jnp.int32)
        ones16 = jnp.ones((16,), jnp.int32)

        def body(j, carry):
            row0 = base_idx_row + j * IDX_ROWS
            pltpu.sync_copy(idx_hbm.at[pl.ds(row0, IDX_ROWS)], idx_v)
            copies = [
                pltpu.async_copy(tbl_hbm.at[idx_v.at[t]],
                                 rows_v.at[pl.ds(t * 128, 128)], sem)
                for t in range(IDX_ROWS)
            ]
            for c in copies:
                c.wait()
            for k in range(CHUNK // 16):
                r = ri + (k * 16)
                c0_v[pl.ds(k * 16, 16)] = plsc.load_gather(rows_v, [r, zeros16])
                c1_v[pl.ds(k * 16, 16)] = plsc.load_gather(rows_v, [r, ones16])
            pltpu.sync_copy(c0_v, out_hbm.at[0, pl.ds(base_out + j * CHUNK, CHUNK)])
            pltpu.sync_copy(c1_v, out_hbm.at[1, pl.ds(base_out + j * CHUNK, CHUNK)])
            return carry

        lax.fori_loop(0, n_outer, body, 0)

    return gather_k(tbl_flat, jnp.asarray(idx_np))


def _level_interp(a_np, g2):
    a = jnp.asarray(a_np.astype(np.float32)).astype(jnp.bfloat16)
    at = jnp.asarray(np.ascontiguousarray(a_np.T).astype(np.float32)).astype(jnp.bfloat16)

    def body(a_ref, at_ref, g_ref, o_ref):
        av = a_ref[...]
        atv = at_ref[...]
        for c in (0, 1):
            t = jnp.dot(g_ref[c], atv, preferred_element_type=jnp.float32)
            e = jnp.dot(av, t.astype(jnp.bfloat16),
                        preferred_element_type=jnp.float32)
            o_ref[c] = e.astype(jnp.bfloat16)

    return pl.pallas_call(
        body,
        out_shape=jax.ShapeDtypeStruct((2, H_RES, W_RES), jnp.bfloat16),
    )(a, at, g2)


def _mlp(planes, w1t, b1, w2t, b2, w3t, b3):
    bn = 16384
    grid = (HW // bn,)

    def body(*refs):
        plane_refs = refs[:L]
        w1_ref, b1_ref, w2_ref, b2_ref, w3_ref, b3_ref, o_ref = refs[L:]
        x = jnp.concatenate([p[...] for p in plane_refs], axis=0)
        h = jnp.dot(w1_ref[...], x, preferred_element_type=jnp.float32)
        h = jnp.maximum(h + b1_ref[...], 0.0).astype(jnp.bfloat16)
        h = jnp.dot(w2_ref[...], h, preferred_element_type=jnp.float32)
        h = jnp.maximum(h + b2_ref[...], 0.0).astype(jnp.bfloat16)
        o = jnp.dot(w3_ref[...], h, preferred_element_type=jnp.float32)
        o_ref[...] = jax.nn.sigmoid(o + b3_ref[...])

    full = lambda s: pl.BlockSpec(s, lambda i: (0, 0))
    return pl.pallas_call(
        body,
        grid=grid,
        in_specs=[pl.BlockSpec((2, bn), lambda i: (0, i)) for _ in range(L)]
        + [
            full((HIDDEN, 32)), full((HIDDEN, 1)),
            full((HIDDEN, HIDDEN)), full((HIDDEN, 1)),
            full((3, HIDDEN)), full((3, 1)),
        ],
        out_specs=pl.BlockSpec((3, bn), lambda i: (0, i)),
        out_shape=jax.ShapeDtypeStruct((3, HW), jnp.float32),
    )(*planes, w1t, b1, w2t, b2, w3t, b3)


def kernel(tables, W1, b1, W2, b2, W3, b3):
    pre = _precompute()
    tbl = tables.reshape(L * T_SIZE, F_DIM)
    g = _sc_gather(tbl, pre["idx"], pre["nv_pad"])

    planes = []
    for l in range(L - 1):
        n, off = pre["ns"][l], pre["offs"][l]
        seg = lax.slice(g, (0, off), (F_DIM, off + n * n))
        g2 = seg.astype(jnp.bfloat16).reshape(F_DIM, n, n)
        planes.append(_level_interp(pre["amats"][l], g2).reshape(F_DIM, HW))
    seg15 = lax.slice(g, (0, pre["off15"]), (F_DIM, pre["off15"] + HW))
    planes.append(seg15.astype(jnp.bfloat16))
    out = _mlp(planes,
               W1.T.astype(jnp.bfloat16), b1.reshape(HIDDEN, 1),
               W2.T.astype(jnp.bfloat16), b2.reshape(HIDDEN, 1),
               W3.T.astype(jnp.bfloat16), b3.reshape(3, 1))
    return out.reshape(1, 3, H_RES, W_RES)


# speedup vs baseline: 169.0815x; 5.0239x over previous
import functools

import numpy as np
import jax
import jax.numpy as jnp
from jax import lax
from jax.experimental import pallas as pl
from jax.experimental.pallas import tpu as pltpu
from jax.experimental.pallas import tpu_sc as plsc

L = 16
W_RES = 1024
H_RES = 1024
HW = W_RES * H_RES
LOG2_T = 19
T_SIZE = 2 ** LOG2_T
F_DIM = 2
HIDDEN = 64
HASH_K = np.uint32(2654435761)
HASH_MASK = np.uint32(T_SIZE - 1)

NC, NS = 2, 16
NW = NC * NS
CHUNK = 2048
IDX_ROWS = CHUNK // 128


def _resolutions_np():
    b = np.exp((np.log(2048.0) - np.log(16.0)) / (L - 1))
    return np.floor(16.0 * (b ** np.arange(L))).astype(np.float32)


@functools.lru_cache(maxsize=1)
def _precompute():
    res = _resolutions_np()
    px = (np.arange(W_RES, dtype=np.float32) / np.float32(W_RES))
    idx_segs, amats, ns, offs = [], [], [], []
    off = 0
    for l in range(L - 1):
        r = np.float32(res[l])
        scaled = (px * r).astype(np.float32)
        pos = np.floor(scaled).astype(np.float32)
        ix = pos.astype(np.uint32)
        fx = (scaled - pos).astype(np.float32)
        n = int(ix.max()) + 2
        g = np.arange(n, dtype=np.uint32)
        hy = (g * HASH_K) & HASH_MASK
        grid = (g[None, :] ^ hy[:, None]) & HASH_MASK
        idx_segs.append((grid.astype(np.int64) + l * T_SIZE)
                        .astype(np.int32).reshape(-1))
        a = np.zeros((W_RES, n), np.float32)
        a[np.arange(W_RES), ix] += (1.0 - fx)
        a[np.arange(W_RES), ix + 1] += fx
        amats.append(a)
        ns.append(n)
        offs.append(off)
        off += n * n
    ix15 = (px * np.float32(res[L - 1])).astype(np.uint32)
    h15 = (ix15[None, :] ^ ((ix15[:, None] * HASH_K) & HASH_MASK)) & HASH_MASK
    idx_segs.append((h15.astype(np.int64) + (L - 1) * T_SIZE)
                    .astype(np.int32).reshape(-1))
    off15 = off
    off += HW
    nv = off
    group = NW * CHUNK
    nv_pad = ((nv + group - 1) // group) * group
    idx = np.concatenate(idx_segs)
    pad = (np.arange(nv_pad - nv, dtype=np.int64) % (L * T_SIZE)).astype(np.int32)
    idx = np.concatenate([idx, pad]).astype(np.int64)
    lvl = idx >> LOG2_T
    t_in = idx & int(HASH_MASK)
    base = lvl * (2 * T_SIZE) + (t_in >> 7) * 256 + (t_in & 127)
    idx0 = base.astype(np.int32)
    idx1 = (base + 128).astype(np.int32)
    idx0 = idx0.reshape(nv_pad // 128, 128)
    idx1 = idx1.reshape(nv_pad // 128, 128)
    return dict(idx0=idx0, idx1=idx1, amats=amats, ns=ns, offs=offs,
                off15=off15, nv_pad=nv_pad)


def _sc_gather(tbl_flat, idx0_np, idx1_np, nv_pad):
    per_w = nv_pad // NW
    n_outer = per_w // CHUNK
    rows_per_w = per_w // 128
    mesh = plsc.VectorSubcoreMesh(core_axis_name="c", subcore_axis_name="s",
                                  num_cores=NC, num_subcores=NS)

    @functools.partial(
        pl.kernel,
        out_type=jax.ShapeDtypeStruct((F_DIM, nv_pad), jnp.float32),
        mesh=mesh,
        scratch_types=[
            pltpu.VMEM((IDX_ROWS, 128), jnp.int32),
            pltpu.VMEM((IDX_ROWS, 128), jnp.int32),
            pltpu.VMEM((CHUNK,), jnp.float32),
            pltpu.VMEM((CHUNK,), jnp.float32),
            pltpu.SemaphoreType.DMA,
        ],
        compiler_params=pltpu.CompilerParams(use_tc_tiling_on_sc=False,
                                             needs_layout_passes=False),
    )
    def gather_k(tbl_hbm, idx0_hbm, idx1_hbm, out_hbm, idx0_v, idx1_v,
                 c0_v, c1_v, sem):
        wid = lax.axis_index("s") * NC + lax.axis_index("c")
        base_idx_row = wid * rows_per_w
        base_out = wid * per_w

        def body(j, carry):
            row0 = base_idx_row + j * IDX_ROWS
            pltpu.sync_copy(idx0_hbm.at[pl.ds(row0, IDX_ROWS)], idx0_v)
            pltpu.sync_copy(idx1_hbm.at[pl.ds(row0, IDX_ROWS)], idx1_v)
            copies = [
                pltpu.async_copy(tbl_hbm.at[iv.at[t]],
                                 cv.at[pl.ds(t * 128, 128)], sem)
                for iv, cv in ((idx0_v, c0_v), (idx1_v, c1_v))
                for t in range(IDX_ROWS)
            ]
            for c in copies:
                c.wait()
            pltpu.sync_copy(c0_v, out_hbm.at[0, pl.ds(base_out + j * CHUNK, CHUNK)])
            pltpu.sync_copy(c1_v, out_hbm.at[1, pl.ds(base_out + j * CHUNK, CHUNK)])
            return carry

        lax.fori_loop(0, n_outer, body, 0)

    return gather_k(tbl_flat, jnp.asarray(idx0_np), jnp.asarray(idx1_np))


def _level_interp(a_np, g2):
    a = jnp.asarray(a_np.astype(np.float32)).astype(jnp.bfloat16)
    at = jnp.asarray(np.ascontiguousarray(a_np.T).astype(np.float32)).astype(jnp.bfloat16)

    def body(a_ref, at_ref, g_ref, o_ref):
        av = a_ref[...]
        atv = at_ref[...]
        for c in (0, 1):
            t = jnp.dot(g_ref[c], atv, preferred_element_type=jnp.float32)
            e = jnp.dot(av, t.astype(jnp.bfloat16),
                        preferred_element_type=jnp.float32)
            o_ref[c] = e.astype(jnp.bfloat16)

    return pl.pallas_call(
        body,
        out_shape=jax.ShapeDtypeStruct((2, H_RES, W_RES), jnp.bfloat16),
    )(a, at, g2)


def _mlp(planes, w1t, b1, w2t, b2, w3t, b3):
    bn = 16384
    grid = (HW // bn,)

    def body(*refs):
        plane_refs = refs[:L]
        w1_ref, b1_ref, w2_ref, b2_ref, w3_ref, b3_ref, o_ref = refs[L:]
        x = jnp.concatenate([p[...] for p in plane_refs], axis=0)
        h = jnp.dot(w1_ref[...], x, preferred_element_type=jnp.float32)
        h = jnp.maximum(h + b1_ref[...], 0.0).astype(jnp.bfloat16)
        h = jnp.dot(w2_ref[...], h, preferred_element_type=jnp.float32)
        h = jnp.maximum(h + b2_ref[...], 0.0).astype(jnp.bfloat16)
        o = jnp.dot(w3_ref[...], h, preferred_element_type=jnp.float32)
        o_ref[...] = jax.nn.sigmoid(o + b3_ref[...])

    full = lambda s: pl.BlockSpec(s, lambda i: (0, 0))
    return pl.pallas_call(
        body,
        grid=grid,
        in_specs=[pl.BlockSpec((2, bn), lambda i: (0, i)) for _ in range(L)]
        + [
            full((HIDDEN, 32)), full((HIDDEN, 1)),
            full((HIDDEN, HIDDEN)), full((HIDDEN, 1)),
            full((3, HIDDEN)), full((3, 1)),
        ],
        out_specs=pl.BlockSpec((3, bn), lambda i: (0, i)),
        out_shape=jax.ShapeDtypeStruct((3, HW), jnp.float32),
    )(*planes, w1t, b1, w2t, b2, w3t, b3)


def kernel(tables, W1, b1, W2, b2, W3, b3):
    pre = _precompute()
    tbl = (tables.reshape(L, T_SIZE // 128, 128, F_DIM)
           .transpose(0, 1, 3, 2).reshape(L * F_DIM * T_SIZE))
    g = _sc_gather(tbl, pre["idx0"], pre["idx1"], pre["nv_pad"])

    planes = []
    for l in range(L - 1):
        n, off = pre["ns"][l], pre["offs"][l]
        seg = lax.slice(g, (0, off), (F_DIM, off + n * n))
        g2 = seg.astype(jnp.bfloat16).reshape(F_DIM, n, n)
        planes.append(_level_interp(pre["amats"][l], g2).reshape(F_DIM, HW))
    seg15 = lax.slice(g, (0, pre["off15"]), (F_DIM, pre["off15"] + HW))
    planes.append(seg15.astype(jnp.bfloat16))
    out = _mlp(planes,
               W1.T.astype(jnp.bfloat16), b1.reshape(HIDDEN, 1),
               W2.T.astype(jnp.bfloat16), b2.reshape(HIDDEN, 1),
               W3.T.astype(jnp.bfloat16), b3.reshape(3, 1))
    return out.reshape(1, 3, H_RES, W_RES)


# speedup vs baseline: 178.4724x; 1.0555x over previous
import functools

import numpy as np
import jax
import jax.numpy as jnp
from jax import lax
from jax.experimental import pallas as pl
from jax.experimental.pallas import tpu as pltpu
from jax.experimental.pallas import tpu_sc as plsc

L = 16
W_RES = 1024
H_RES = 1024
HW = W_RES * H_RES
LOG2_T = 19
T_SIZE = 2 ** LOG2_T
F_DIM = 2
HIDDEN = 64
HASH_K = np.uint32(2654435761)
HASH_MASK = np.uint32(T_SIZE - 1)

NC, NS = 2, 16
NW = NC * NS
CHUNK = 2048
IDX_ROWS = CHUNK // 128


def _resolutions_np():
    b = np.exp((np.log(2048.0) - np.log(16.0)) / (L - 1))
    return np.floor(16.0 * (b ** np.arange(L))).astype(np.float32)


@functools.lru_cache(maxsize=1)
def _precompute():
    res = _resolutions_np()
    px = (np.arange(W_RES, dtype=np.float32) / np.float32(W_RES))
    idx_segs, amats, ns, npads, offs = [], [], [], [], []
    off = 0
    for l in range(L - 1):
        r = np.float32(res[l])
        scaled = (px * r).astype(np.float32)
        pos = np.floor(scaled).astype(np.float32)
        ix = pos.astype(np.uint32)
        fx = (scaled - pos).astype(np.float32)
        n = int(ix.max()) + 2
        g = np.arange(n, dtype=np.uint32)
        hy = (g * HASH_K) & HASH_MASK
        grid = (g[None, :] ^ hy[:, None]) & HASH_MASK
        npad = ((n + 127) // 128) * 128
        gridp = np.empty((n, npad), np.uint32)
        for row in range(n):
            gridp[row] = np.resize(grid[row], npad)
        seg = (gridp.astype(np.int64) + l * T_SIZE).astype(np.int32).reshape(-1)
        seg_len = ((seg.size + CHUNK - 1) // CHUNK) * CHUNK
        seg = np.resize(seg, seg_len)
        idx_segs.append(seg)
        a = np.zeros((W_RES, n), np.float32)
        a[np.arange(W_RES), ix] += (1.0 - fx)
        a[np.arange(W_RES), ix + 1] += fx
        amats.append(a)
        ns.append(n)
        npads.append(npad)
        offs.append(off)
        off += seg_len
    ix15 = (px * np.float32(res[L - 1])).astype(np.uint32)
    h15 = (ix15[None, :] ^ ((ix15[:, None] * HASH_K) & HASH_MASK)) & HASH_MASK
    idx_segs.append((h15.astype(np.int64) + (L - 1) * T_SIZE)
                    .astype(np.int32).reshape(-1))
    off15 = off
    off += HW
    nv = off
    group = NW * CHUNK
    nv_pad = ((nv + group - 1) // group) * group
    idx = np.concatenate(idx_segs)
    pad = (np.arange(nv_pad - nv, dtype=np.int64) % (L * T_SIZE)).astype(np.int32)
    idx = np.concatenate([idx, pad]).astype(np.int64)
    lvl = idx >> LOG2_T
    t_in = idx & int(HASH_MASK)
    base = lvl * (2 * T_SIZE) + (t_in >> 7) * 256 + (t_in & 127)
    idx0 = base.astype(np.int32)
    idx1 = (base + 128).astype(np.int32)
    return dict(idx0=idx0, idx1=idx1, amats=amats, ns=ns, npads=npads,
                offs=offs, off15=off15, nv_pad=nv_pad)


def _sc_gather(tbl_flat, idx0_np, idx1_np, nv_pad):
    per_w = nv_pad // NW
    n_outer = per_w // CHUNK
    mesh = plsc.VectorSubcoreMesh(core_axis_name="c", subcore_axis_name="s",
                                  num_cores=NC, num_subcores=NS)

    @functools.partial(
        pl.kernel,
        out_type=jax.ShapeDtypeStruct((F_DIM, nv_pad), jnp.float32),
        mesh=mesh,
        scratch_types=[
            pltpu.VMEM((CHUNK,), jnp.int32),
            pltpu.VMEM((CHUNK,), jnp.int32),
            pltpu.VMEM((CHUNK,), jnp.float32),
            pltpu.VMEM((CHUNK,), jnp.float32),
            pltpu.SemaphoreType.DMA,
        ],
        compiler_params=pltpu.CompilerParams(use_tc_tiling_on_sc=False,
                                             needs_layout_passes=False),
    )
    def gather_k(tbl_hbm, idx0_hbm, idx1_hbm, out_hbm, idx0_v, idx1_v,
                 c0_v, c1_v, sem):
        wid = lax.axis_index("s") * NC + lax.axis_index("c")
        base_out = wid * per_w

        def body(j, carry):
            base = base_out + j * CHUNK
            pltpu.sync_copy(idx0_hbm.at[pl.ds(base, CHUNK)], idx0_v)
            pltpu.sync_copy(idx1_hbm.at[pl.ds(base, CHUNK)], idx1_v)
            copies = [
                pltpu.async_copy(tbl_hbm.at[iv.at[pl.ds(t * 128, 128)]],
                                 cv.at[pl.ds(t * 128, 128)], sem)
                for iv, cv in ((idx0_v, c0_v), (idx1_v, c1_v))
                for t in range(IDX_ROWS)
            ]
            for c in copies:
                c.wait()
            pltpu.sync_copy(c0_v, out_hbm.at[0, pl.ds(base, CHUNK)])
            pltpu.sync_copy(c1_v, out_hbm.at[1, pl.ds(base, CHUNK)])
            return carry

        lax.fori_loop(0, n_outer, body, 0)

    return gather_k(tbl_flat, jnp.asarray(idx0_np), jnp.asarray(idx1_np))


def _level_interp(a_np, npad, g2):
    n = a_np.shape[1]
    a = jnp.asarray(a_np).astype(jnp.bfloat16)
    atp_np = np.zeros((npad, W_RES), np.float32)
    atp_np[:n] = a_np.T
    at = jnp.asarray(atp_np).astype(jnp.bfloat16)

    def body(a_ref, at_ref, g_ref, o_ref):
        av = a_ref[...]
        atv = at_ref[...]
        for c in (0, 1):
            t = jnp.dot(g_ref[c], atv, preferred_element_type=jnp.float32)
            e = jnp.dot(av, t.astype(jnp.bfloat16),
                        preferred_element_type=jnp.float32)
            o_ref[c] = e.astype(jnp.bfloat16)

    return pl.pallas_call(
        body,
        out_shape=jax.ShapeDtypeStruct((2, H_RES, W_RES), jnp.bfloat16),
    )(a, at, g2)


def _mlp(planes, w1t, b1, w2t, b2, w3t, b3):
    bn = 16384
    grid = (HW // bn,)

    def body(*refs):
        plane_refs = refs[:L]
        w1_ref, b1_ref, w2_ref, b2_ref, w3_ref, b3_ref, o_ref = refs[L:]
        x = jnp.concatenate([p[...] for p in plane_refs], axis=0)
        h = jnp.dot(w1_ref[...], x, preferred_element_type=jnp.float32)
        h = jnp.maximum(h + b1_ref[...], 0.0).astype(jnp.bfloat16)
        h = jnp.dot(w2_ref[...], h, preferred_element_type=jnp.float32)
        h = jnp.maximum(h + b2_ref[...], 0.0).astype(jnp.bfloat16)
        o = jnp.dot(w3_ref[...], h, preferred_element_type=jnp.float32)
        o_ref[...] = jax.nn.sigmoid(o + b3_ref[...])

    full = lambda s: pl.BlockSpec(s, lambda i: (0, 0))
    return pl.pallas_call(
        body,
        grid=grid,
        in_specs=[pl.BlockSpec((2, bn), lambda i: (0, i)) for _ in range(L)]
        + [
            full((HIDDEN, 32)), full((HIDDEN, 1)),
            full((HIDDEN, HIDDEN)), full((HIDDEN, 1)),
            full((3, HIDDEN)), full((3, 1)),
        ],
        out_specs=pl.BlockSpec((3, bn), lambda i: (0, i)),
        out_shape=jax.ShapeDtypeStruct((3, HW), jnp.float32),
    )(*planes, w1t, b1, w2t, b2, w3t, b3)


def kernel(tables, W1, b1, W2, b2, W3, b3):
    pre = _precompute()
    tbl = (tables.reshape(L, T_SIZE // 128, 128, F_DIM)
           .transpose(0, 1, 3, 2).reshape(L * F_DIM * T_SIZE))
    g = _sc_gather(tbl, pre["idx0"], pre["idx1"], pre["nv_pad"])

    gb = g.astype(jnp.bfloat16)
    planes = []
    for l in range(L - 1):
        n, npad, off = pre["ns"][l], pre["npads"][l], pre["offs"][l]
        seg = lax.slice(gb, (0, off), (F_DIM, off + n * npad))
        g2 = seg.reshape(F_DIM, n, npad)
        planes.append(_level_interp(pre["amats"][l], npad, g2)
                      .reshape(F_DIM, HW))
    seg15 = lax.slice(gb, (0, pre["off15"]), (F_DIM, pre["off15"] + HW))
    planes.append(seg15)
    out = _mlp(planes,
               W1.T.astype(jnp.bfloat16), b1.reshape(HIDDEN, 1),
               W2.T.astype(jnp.bfloat16), b2.reshape(HIDDEN, 1),
               W3.T.astype(jnp.bfloat16), b3.reshape(3, 1))
    return out.reshape(1, 3, H_RES, W_RES)
